# baseline (speedup 1.0000x reference)
import functools

import jax
import jax.numpy as jnp
from jax import lax
from jax.experimental import pallas as pl
from jax.experimental.pallas import tpu as pltpu
from jax.experimental.pallas import tpu_sc as plsc

_H = 512
_W = 512
_C = 192
_K = 500
_KPAD = 512
_NT = 128
_PLANE_HW = _H * _W
def _detect_body(planes_ref, scores_ref, kx_ref, ky_ref, pos_ref):
    r = lax.broadcasted_iota(jnp.int32, (_NT, _NT), 0)
    c = lax.broadcasted_iota(jnp.int32, (_NT, _NT), 1)
    best_v = jnp.full((_NT, _NT), -1.0, dtype=jnp.float32)
    best_k = jnp.zeros((_NT, _NT), dtype=jnp.int32)
    for p in range(16):
        dy, dx = p // 4, p % 4
        y = 4 * r + dy
        x = 4 * c + dx
        v = planes_ref[p]
        mask = (y >= 3) & (y <= _H - 4) & (x >= 3) & (x <= _W - 4)
        v = jnp.where(mask, v, 0.0)
        upd = v > best_v
        best_v = jnp.where(upd, v, best_v)
        best_k = jnp.where(upd, p, best_k)
    intmax = jnp.int32(2147483647)
    yy = 4 * r + best_k // 4
    xx = 4 * c + best_k % 4
    phys = ((yy // 8) * 4 + xx // 128) * 1024 + (yy % 8) * 128 + (xx % 128)
    flat = r * _NT + c
    packed = (flat - 8192) * (1 << 18) + phys

    rio4 = lax.broadcasted_iota(jnp.int32, (4, _NT), 0)
    cio4 = lax.broadcasted_iota(jnp.int32, (4, _NT), 1)

    def lane_allmax(t):
        for d in (1, 2, 4, 8, 16, 32, 64):
            t = jnp.maximum(t, pltpu.roll(t, d, 1))
        return t

    def lane_allmin(t):
        for d in (1, 2, 4, 8, 16, 32, 64):
            t = jnp.minimum(t, pltpu.roll(t, d, 1))
        return t

    def body(k, carry):
        v, sacc, pacc = carry
        m = lane_allmax(jnp.max(v, axis=0, keepdims=True))
        sel = lane_allmin(
            jnp.min(jnp.where(v == m, packed, intmax), axis=0, keepdims=True)
        )
        mask_k = (rio4 == k // _NT) & (cio4 == k % _NT)
        sacc = jnp.where(mask_k, m, sacc)
        pacc = jnp.where(mask_k, sel, pacc)
        v = jnp.where(packed == sel, -1.0, v)
        return (v, sacc, pacc)

    _, sacc, pacc = lax.fori_loop(
        0,
        _KPAD,
        body,
        (
            best_v,
            jnp.zeros((4, _NT), jnp.float32),
            jnp.zeros((4, _NT), jnp.int32),
        ),
    )

    posv = pacc & ((1 << 18) - 1)
    t = posv >> 10
    inr = posv & 1023
    yi = (t >> 2) * 8 + (inr >> 7)
    xi = (t & 3) * 128 + (inr & 127)
    scores_ref[...] = sacc
    kx_ref[...] = xi.astype(jnp.float32) / (_W - 1) * 2.0 - 1.0
    ky_ref[...] = yi.astype(jnp.float32) / (_H - 1) * 2.0 - 1.0
    pos_ref[...] = posv


def _detect(planes):
    return pl.pallas_call(
        _detect_body,
        in_specs=[pl.BlockSpec(memory_space=pltpu.VMEM)],
        out_specs=[
            pl.BlockSpec(memory_space=pltpu.VMEM),
            pl.BlockSpec(memory_space=pltpu.VMEM),
            pl.BlockSpec(memory_space=pltpu.VMEM),
            pl.BlockSpec(memory_space=pltpu.VMEM),
        ],
        out_shape=[
            jax.ShapeDtypeStruct((4, _NT), jnp.float32),
            jax.ShapeDtypeStruct((4, _NT), jnp.float32),
            jax.ShapeDtypeStruct((4, _NT), jnp.float32),
            jax.ShapeDtypeStruct((4, _NT), jnp.int32),
        ],
    )(planes)


@functools.cache
def _make_gather_norm():
    mesh = plsc.VectorSubcoreMesh(core_axis_name="c", subcore_axis_name="s")
    n_per_w = 16 * _C

    @functools.partial(
        pl.kernel,
        mesh=mesh,
        out_type=jax.ShapeDtypeStruct((_KPAD * _C,), jnp.float32),
        scratch_types=[
            pltpu.VMEM((16,), jnp.int32),
            pltpu.VMEM((n_per_w,), jnp.int32),
            pltpu.VMEM((n_per_w,), jnp.float32),
            pltpu.SemaphoreType.DMA,
        ],
    )
    def gather_norm(pos_hbm, dm_hbm, out_hbm, pos_v, idx_v, rows_v, sem):
        wid = lax.axis_index("s") * 2 + lax.axis_index("c")
        base = wid * n_per_w
        pltpu.sync_copy(pos_hbm.at[pl.ds(wid * 16, 16)], pos_v)
        pos_vec = pos_v[...]
        for c in range(_C):
            idx_v[pl.ds(c * 16, 16)] = pos_vec + c * _PLANE_HW

        def chunk(j, carry):
            sl = pl.ds(j * 128, 128)
            pltpu.async_copy(dm_hbm.at[idx_v.at[sl]], rows_v.at[sl], sem).wait()
            return carry

        lax.fori_loop(0, n_per_w // 128, chunk, 0)

        pltpu.sync_copy(rows_v, out_hbm.at[pl.ds(base, n_per_w)])

    return gather_norm


def _norm_body(d_ref, out_ref):
    d = d_ref[...]
    n = jnp.sqrt(jnp.sum(d * d, axis=1, keepdims=True))
    out_ref[...] = d / n


def _normalize(desc):
    return pl.pallas_call(
        _norm_body,
        in_specs=[pl.BlockSpec(memory_space=pltpu.VMEM)],
        out_specs=pl.BlockSpec(memory_space=pltpu.VMEM),
        out_shape=jax.ShapeDtypeStruct((_KPAD, _C), jnp.float32),
    )(desc)


@jax.jit
def kernel(scores_map, descriptor_map):
    scores = scores_map[0, 0]
    planes = (
        scores.reshape(_NT, 4, _NT, 4).transpose(1, 3, 0, 2).reshape(16, _NT, _NT)
    )
    kptscores, kx, ky, pos = _detect(planes)
    kptscores = kptscores.reshape(_KPAD)
    keypoints = jnp.stack([kx.reshape(_KPAD), ky.reshape(_KPAD)], axis=1)
    pos = pos.reshape(_KPAD)
    dm_flat = (
        descriptor_map.reshape(_C, _H // 8, 8, _W // 128, 128)
        .transpose(0, 1, 3, 2, 4)
        .reshape(-1)
    )
    desc = _make_gather_norm()(pos, dm_flat)
    desc = desc.reshape(32, _C, 16).transpose(0, 2, 1).reshape(_KPAD, _C)
    desc = _normalize(desc)
    return (keypoints[:_K], desc[:_K], kptscores[:_K])


# speedup vs baseline: 2.2272x; 2.2272x over previous
import functools

import jax
import jax.numpy as jnp
from jax import lax
from jax.experimental import pallas as pl
from jax.experimental.pallas import tpu as pltpu
from jax.experimental.pallas import tpu_sc as plsc

_H = 512
_W = 512
_C = 192
_K = 500
_KPAD = 512
_NT = 128
_PLANE_HW = _H * _W
def _detect_body(planes_ref, scores_ref, kx_ref, ky_ref, pos_ref):
    r = lax.broadcasted_iota(jnp.int32, (_NT, _NT), 0)
    c = lax.broadcasted_iota(jnp.int32, (_NT, _NT), 1)
    best_v = jnp.full((_NT, _NT), -1.0, dtype=jnp.float32)
    best_k = jnp.zeros((_NT, _NT), dtype=jnp.int32)
    for p in range(16):
        dy, dx = p // 4, p % 4
        y = 4 * r + dy
        x = 4 * c + dx
        v = planes_ref[p]
        mask = (y >= 3) & (y <= _H - 4) & (x >= 3) & (x <= _W - 4)
        v = jnp.where(mask, v, 0.0)
        upd = v > best_v
        best_v = jnp.where(upd, v, best_v)
        best_k = jnp.where(upd, p, best_k)
    intmax = jnp.int32(2147483647)
    yy = 4 * r + best_k // 4
    xx = 4 * c + best_k % 4
    phys = ((yy // 8) * 4 + xx // 128) * 1024 + (yy % 8) * 128 + (xx % 128)
    flat = r * _NT + c
    packed = (flat - 8192) * (1 << 18) + phys

    def body(k, v):
        m = jnp.max(v)
        sel = jnp.min(jnp.where(v == m, packed, intmax))
        posv = sel & ((1 << 18) - 1)
        t = posv >> 10
        inr = posv & 1023
        yi = (t >> 2) * 8 + (inr >> 7)
        xi = (t & 3) * 128 + (inr & 127)
        scores_ref[k] = m
        kx_ref[k] = xi.astype(jnp.float32) / (_W - 1) * 2.0 - 1.0
        ky_ref[k] = yi.astype(jnp.float32) / (_H - 1) * 2.0 - 1.0
        pos_ref[k] = posv
        return jnp.where(packed == sel, -1.0, v)

    lax.fori_loop(0, _KPAD, body, best_v)


def _detect(planes):
    return pl.pallas_call(
        _detect_body,
        in_specs=[pl.BlockSpec(memory_space=pltpu.VMEM)],
        out_specs=[
            pl.BlockSpec(memory_space=pltpu.SMEM),
            pl.BlockSpec(memory_space=pltpu.SMEM),
            pl.BlockSpec(memory_space=pltpu.SMEM),
            pl.BlockSpec(memory_space=pltpu.SMEM),
        ],
        out_shape=[
            jax.ShapeDtypeStruct((_KPAD,), jnp.float32),
            jax.ShapeDtypeStruct((_KPAD,), jnp.float32),
            jax.ShapeDtypeStruct((_KPAD,), jnp.float32),
            jax.ShapeDtypeStruct((_KPAD,), jnp.int32),
        ],
    )(planes)


@functools.cache
def _make_gather_norm():
    mesh = plsc.VectorSubcoreMesh(core_axis_name="c", subcore_axis_name="s")
    n_per_w = 16 * _C

    @functools.partial(
        pl.kernel,
        mesh=mesh,
        out_type=jax.ShapeDtypeStruct((_KPAD * _C,), jnp.float32),
        scratch_types=[
            pltpu.VMEM((16,), jnp.int32),
            pltpu.VMEM((n_per_w,), jnp.int32),
            pltpu.VMEM((n_per_w,), jnp.float32),
            pltpu.SemaphoreType.DMA,
        ],
    )
    def gather_norm(pos_hbm, dm_hbm, out_hbm, pos_v, idx_v, rows_v, sem):
        wid = lax.axis_index("s") * 2 + lax.axis_index("c")
        base = wid * n_per_w
        pltpu.sync_copy(pos_hbm.at[pl.ds(wid * 16, 16)], pos_v)
        pos_vec = pos_v[...]
        for c in range(_C):
            idx_v[pl.ds(c * 16, 16)] = pos_vec + c * _PLANE_HW

        def chunk(j, carry):
            sl = pl.ds(j * 128, 128)
            pltpu.async_copy(dm_hbm.at[idx_v.at[sl]], rows_v.at[sl], sem).wait()
            return carry

        lax.fori_loop(0, n_per_w // 128, chunk, 0)

        pltpu.sync_copy(rows_v, out_hbm.at[pl.ds(base, n_per_w)])

    return gather_norm


def _norm_body(d_ref, out_ref):
    d = d_ref[...]
    n = jnp.sqrt(jnp.sum(d * d, axis=1, keepdims=True))
    out_ref[...] = d / n


def _normalize(desc):
    return pl.pallas_call(
        _norm_body,
        in_specs=[pl.BlockSpec(memory_space=pltpu.VMEM)],
        out_specs=pl.BlockSpec(memory_space=pltpu.VMEM),
        out_shape=jax.ShapeDtypeStruct((_KPAD, _C), jnp.float32),
    )(desc)


@jax.jit
def kernel(scores_map, descriptor_map):
    scores = scores_map[0, 0]
    planes = (
        scores.reshape(_NT, 4, _NT, 4).transpose(1, 3, 0, 2).reshape(16, _NT, _NT)
    )
    kptscores, kx, ky, pos = _detect(planes)
    keypoints = jnp.stack([kx, ky], axis=1)
    dm_flat = (
        descriptor_map.reshape(_C, _H // 8, 8, _W // 128, 128)
        .transpose(0, 1, 3, 2, 4)
        .reshape(-1)
    )
    desc = _make_gather_norm()(pos, dm_flat)
    desc = desc.reshape(32, _C, 16).transpose(0, 2, 1).reshape(_KPAD, _C)
    desc = _normalize(desc)
    return (keypoints[:_K], desc[:_K], kptscores[:_K])
